# baseline (speedup 1.0000x reference)
import functools

import jax
import jax.numpy as jnp
from jax import lax
from jax.experimental import pallas as pl
from jax.experimental.pallas import tpu as pltpu
from jax.experimental.pallas import tpu_sc as plsc

B, S, D = 16, 2048, 1024
DH = D // 2
K = 32
NV = DH // 16
NT = 16
RING = 4
AHEAD = RING - 1
NCHMAX = B * 4 + RING

_mesh = plsc.VectorSubcoreMesh(core_axis_name="c", subcore_axis_name="s")


@functools.partial(
    pl.kernel,
    out_type=jax.ShapeDtypeStruct((B, D), jnp.float32),
    mesh=_mesh,
    scratch_types=[
        pltpu.VMEM((RING, K, DH), jnp.float32),
        pltpu.VMEM((B * DH,), jnp.float32),
        pltpu.VMEM((32,), jnp.int32),
        pltpu.VMEM((NT, DH), jnp.float32),
        pltpu.VMEM((DH,), jnp.float32),
        pltpu.VMEM_SHARED((NT, B * DH), jnp.float32),
        pltpu.SMEM((4, NCHMAX), jnp.int32),
        pltpu.SemaphoreType.DMA,
        pltpu.SemaphoreType.DMA,
        pltpu.SemaphoreType.DMA,
        pltpu.SemaphoreType.DMA,
    ],
)
def _sc_mean(x_hbm, len_hbm, out_hbm, buf, acc, lenv, redbuf, outb, shared,
             desc, *sems):
    c = lax.axis_index("c")
    s = lax.axis_index("s")
    dh0 = c * DH
    pltpu.sync_copy(len_hbm, lenv.at[pl.ds(0, 16)])
    zero = jnp.zeros((16,), jnp.float32)

    def build_b(b, g):
        len_b = lenv[pl.ds(b, 16)][0]
        q = ((len_b + NT * 8 - 1) // (NT * 8)) * 8
        start = s * q
        cnt = jnp.clip(len_b - start, 0, q)
        nch = (cnt + K - 1) // K

        def build_j(j, g2):
            raw = start + j * K
            dstart = jnp.minimum(raw, S - K)
            d = raw - dstart
            rmax = jnp.minimum(K, cnt - j * K)
            desc[0, g2] = b
            desc[1, g2] = dstart
            desc[2, g2] = d
            desc[3, g2] = d + rmax
            return g2 + 1

        return lax.fori_loop(0, nch, build_j, g)

    nch_tot = lax.fori_loop(0, B, build_b, jnp.int32(0))

    nround = (nch_tot + RING - 1) // RING
    padded = nround * RING
    last_b = desc[0, jnp.maximum(nch_tot - 1, 0)]

    def pad_p(p, carry):
        g2 = nch_tot + p
        desc[0, g2] = last_b
        desc[1, g2] = 0
        desc[2, g2] = 0
        desc[3, g2] = 0
        return carry

    lax.fori_loop(0, padded - nch_tot, pad_p, jnp.int32(0))

    def zero_b(b, carry):
        for v in range(NV):
            acc[pl.ds(b * DH + v * 16, 16)] = zero
        return carry

    lax.fori_loop(0, B, zero_b, jnp.int32(0))

    def issue(g, slot):
        dstart = pl.multiple_of(desc[1, g], 8)
        src = x_hbm.at[desc[0, g], pl.ds(dstart, K), pl.ds(dh0, DH)]
        pltpu.async_copy(src, buf.at[slot], sems[slot])

    def wait_slot(slot):
        pltpu.make_async_copy(
            x_hbm.at[0, pl.ds(0, K), pl.ds(0, DH)],
            buf.at[slot], sems[slot]).wait()

    for gp in range(AHEAD):
        @pl.when(gp < padded)
        def _(gp=gp):
            issue(jnp.int32(gp), gp % RING)

    def round_body(rd, carry):
        g0 = rd * RING
        bprev = carry[0]
        accs = carry[1:]
        for r in range(RING):
            g = g0 + r
            b = desc[0, g]
            d0 = desc[2, g]
            d1 = desc[3, g]

            @pl.when(b != bprev)
            def _flush(bprev=bprev, accs=accs):
                for v in range(NV):
                    acc[pl.ds(bprev * DH + v * 16, 16)] = accs[v]

            keep = (b == bprev).astype(jnp.float32)
            accs = tuple(a * keep for a in accs)

            wait_slot(r)

            @pl.when(g + AHEAD < padded)
            def _issue_next(g=g, r=r):
                issue(g + AHEAD, (r + AHEAD) % RING)

            def row(i, a, r=r):
                return tuple(a[v] + buf[r, i, pl.ds(v * 16, 16)]
                             for v in range(NV))

            accs = lax.fori_loop(d0, d1, row, accs)
            bprev = b
        return (bprev,) + accs

    first_b = desc[0, 0]
    final = lax.fori_loop(0, nround, round_body, (first_b,) + (zero,) * NV)

    @pl.when(nch_tot > 0)
    def _final_flush():
        blast = final[0]
        for v in range(NV):
            acc[pl.ds(blast * DH + v * 16, 16)] = final[1 + v]

    pltpu.sync_copy(acc, shared.at[s])
    plsc.subcore_barrier()
    pltpu.sync_copy(shared.at[pl.ds(0, NT), pl.ds(s * DH, DH)], redbuf)

    def red(t, a):
        return tuple(a[v] + redbuf[t, pl.ds(v * 16, 16)]
                     for v in range(NV))

    tot = lax.fori_loop(0, NT, red, (zero,) * NV)
    len_s = lenv[pl.ds(s, 16)][0]
    inv_v = jnp.full((16,), 1.0, jnp.float32) / len_s.astype(jnp.float32)
    for v in range(NV):
        outb[pl.ds(v * 16, 16)] = tot[v] * inv_v
    pltpu.sync_copy(outb, out_hbm.at[s, pl.ds(dh0, DH)])


def kernel(input, data_mask, length):
    del data_mask
    return _sc_mean(input, length.astype(jnp.int32))
